# baseline (speedup 1.0000x reference)
import functools

import numpy as np
import jax
import jax.numpy as jnp
from jax.experimental import pallas as pl
from jax.experimental.pallas import tpu as pltpu

_EPS = 1e-5


def _tap_masks_hw(H, W):
    hh = np.arange(H)[:, None]
    ww = np.arange(W)[None, :]
    cols = []
    for kh in range(3):
        for kw in range(3):
            dh, dw = kh - 1, kw - 1
            m = ((hh + dh >= 0) & (hh + dh < H) &
                 (ww + dw >= 0) & (ww + dw < W))
            cols.append(m.reshape(-1))
    return np.stack(cols, axis=1).astype(np.float32)


def _linker_kernel(W, D, Cmid, Dt, x_ref, w1_ref, bn2_ref, mask_ref, w3_ref,
                   bn3_ref, o_ref, t_ref, o3_ref):
    j = pl.program_id(1)
    HW = x_ref.shape[1]

    @pl.when(j == 0)
    def _compute():
        s2 = bn2_ref[0:1] * jax.lax.rsqrt(bn2_ref[3:4] + _EPS)
        b2 = bn2_ref[1:2] - bn2_ref[2:3] * s2
        y = jnp.maximum(
            jnp.dot(x_ref[0], w1_ref[...],
                    preferred_element_type=jnp.float32) * s2 + b2, 0.0)
        yb = y.astype(jnp.bfloat16)
        for kh in range(3):
            for kw in range(3):
                t = kh * 3 + kw
                off = (kh - 1) * W + (kw - 1)
                if off == 0:
                    shifted = yb
                else:
                    shifted = pltpu.roll(yb, shift=(-off) % HW, axis=0)
                    shifted = shifted * mask_ref[:, t:t + 1]
                t_ref[:, t * Cmid:(t + 1) * Cmid] = shifted
        tv = t_ref[...]
        p0 = jnp.dot(tv, w3_ref[0], preferred_element_type=jnp.float32)
        p1 = jnp.dot(tv, w3_ref[1], preferred_element_type=jnp.float32)
        p2 = jnp.dot(tv, w3_ref[2], preferred_element_type=jnp.float32)
        sv = bn3_ref[0:1] * jax.lax.rsqrt(bn3_ref[3:4] + _EPS)
        bv = bn3_ref[1:2] - bn3_ref[2:3] * sv
        o3_ref[:, 0:Cmid] = jnp.maximum((p1 + p2) * sv + bv, 0.0)
        o3_ref[:, Cmid:2 * Cmid] = jnp.maximum(
            (p0 + p1 + p2) * sv + bv, 0.0)
        o3_ref[:, 2 * Cmid:3 * Cmid] = jnp.maximum((p0 + p1) * sv + bv, 0.0)

    o_ref[0] = jnp.broadcast_to(o3_ref[:, Cmid:2 * Cmid][None],
                                (Dt, HW, Cmid))

    @pl.when(j == 0)
    def _front():
        o_ref[0, 0] = o3_ref[:, 0:Cmid]

    @pl.when(j == D // Dt - 1)
    def _back():
        o_ref[0, Dt - 1] = o3_ref[:, 2 * Cmid:3 * Cmid]


def _linker(x_nchw, w1, bn2d, w3, bn3d):
    N, Cin, H, Wd = x_nchw.shape
    Cmid = w1.shape[1]
    D = H
    HW = H * Wd

    bn2 = jnp.stack(bn2d, axis=0)
    bn3 = jnp.stack(bn3d, axis=0)

    w3r = w3.reshape(3, 9 * Cmid, Cmid).astype(jnp.bfloat16)

    x3 = x_nchw.transpose(0, 2, 3, 1).reshape(N, HW, Cin).astype(jnp.float32)
    masks = jnp.asarray(_tap_masks_hw(H, Wd)).astype(jnp.bfloat16)

    Dt = 16
    while D % Dt:
        Dt //= 2

    out = pl.pallas_call(
        functools.partial(_linker_kernel, Wd, D, Cmid, Dt),
        out_shape=jax.ShapeDtypeStruct((N, D, HW, Cmid), jnp.float32),
        grid=(N, D // Dt),
        in_specs=[
            pl.BlockSpec((1, HW, Cin), lambda n, d: (n, 0, 0)),
            pl.BlockSpec((Cin, Cmid), lambda n, d: (0, 0)),
            pl.BlockSpec((4, Cmid), lambda n, d: (0, 0)),
            pl.BlockSpec((HW, 9), lambda n, d: (0, 0)),
            pl.BlockSpec((3, 9 * Cmid, Cmid), lambda n, d: (0, 0, 0)),
            pl.BlockSpec((4, Cmid), lambda n, d: (0, 0)),
        ],
        out_specs=pl.BlockSpec((1, Dt, HW, Cmid), lambda n, d: (n, d, 0, 0)),
        scratch_shapes=[
            pltpu.VMEM((HW, 9 * Cmid), jnp.bfloat16),
            pltpu.VMEM((HW, 3 * Cmid), jnp.float32),
        ],
        compiler_params=pltpu.CompilerParams(
            dimension_semantics=("arbitrary", "arbitrary")),
    )(x3, w1.astype(jnp.float32), bn2, masks, w3r, bn3)

    return out.reshape(N, D, H, Wd, Cmid).transpose(0, 4, 1, 2, 3)


def _base_kernel(x_ref, w_ref, b_ref, o_ref):
    acc = jnp.dot(x_ref[...], w_ref[...], preferred_element_type=jnp.float32)
    o_ref[...] = jnp.maximum(acc + b_ref[...], 0.0)


def _base_matmul(x, w, b, tn=4096):
    M, K = x.shape
    K2, Nc = w.shape
    assert K == K2
    tn = min(tn, Nc)
    if Nc % tn != 0:
        tn = Nc
    b2 = jnp.reshape(b, (1, Nc)).astype(jnp.float32)
    return pl.pallas_call(
        _base_kernel,
        out_shape=jax.ShapeDtypeStruct((M, Nc), jnp.float32),
        grid=(Nc // tn,),
        in_specs=[
            pl.BlockSpec((M, K), lambda j: (0, 0)),
            pl.BlockSpec((K, tn), lambda j: (0, j)),
            pl.BlockSpec((1, tn), lambda j: (0, j)),
        ],
        out_specs=pl.BlockSpec((M, tn), lambda j: (0, j)),
        compiler_params=pltpu.CompilerParams(
            dimension_semantics=("parallel",)),
    )(x.astype(jnp.float32), w.astype(jnp.float32), b2)


def kernel(l0_w1, l0_bn2d_gamma, l0_bn2d_beta, l0_bn2d_mean, l0_bn2d_var,
           l0_w3, l0_bn3d_gamma, l0_bn3d_beta, l0_bn3d_mean, l0_bn3d_var,
           l1_w1, l1_bn2d_gamma, l1_bn2d_beta, l1_bn2d_mean, l1_bn2d_var,
           l1_w3, l1_bn3d_gamma, l1_bn3d_beta, l1_bn3d_mean, l1_bn3d_var,
           base_w, base_b, feat0, feat1, final_vector):
    out0 = _linker(feat0, l0_w1,
                   (l0_bn2d_gamma, l0_bn2d_beta, l0_bn2d_mean, l0_bn2d_var),
                   l0_w3,
                   (l0_bn3d_gamma, l0_bn3d_beta, l0_bn3d_mean, l0_bn3d_var))
    out1 = _linker(feat1, l1_w1,
                   (l1_bn2d_gamma, l1_bn2d_beta, l1_bn2d_mean, l1_bn2d_var),
                   l1_w3,
                   (l1_bn3d_gamma, l1_bn3d_beta, l1_bn3d_mean, l1_bn3d_var))
    N = final_vector.shape[0]
    flat = final_vector.reshape(N, -1)
    x = _base_matmul(flat, base_w, base_b)
    return x, [out0, out1]


# speedup vs baseline: 1.0896x; 1.0896x over previous
import functools

import numpy as np
import jax
import jax.numpy as jnp
from jax.experimental import pallas as pl
from jax.experimental.pallas import tpu as pltpu

_EPS = 1e-5


def _tap_masks_hw(H, W):
    hh = np.arange(H)[:, None]
    ww = np.arange(W)[None, :]
    cols = []
    for kh in range(3):
        for kw in range(3):
            dh, dw = kh - 1, kw - 1
            m = ((hh + dh >= 0) & (hh + dh < H) &
                 (ww + dw >= 0) & (ww + dw < W))
            cols.append(m.reshape(-1))
    return np.stack(cols, axis=1).astype(np.float32)


def _linker_kernel(W, D, Cmid, Dt, x_ref, w1_ref, bn2_ref, mask_ref, w3_ref,
                   bn3_ref, o_ref, t_ref, w3b_ref, o3_ref):
    j = pl.program_id(1)
    HW = x_ref.shape[1]

    @pl.when(j == 0)
    def _convert():
        w3b_ref[...] = w3_ref[...].astype(jnp.bfloat16)

    @pl.when(j == 0)
    def _compute():
        s2 = bn2_ref[0:1] * jax.lax.rsqrt(bn2_ref[3:4] + _EPS)
        b2 = bn2_ref[1:2] - bn2_ref[2:3] * s2
        y = jnp.maximum(
            jnp.dot(x_ref[0], w1_ref[...],
                    preferred_element_type=jnp.float32) * s2 + b2, 0.0)
        yb = y.astype(jnp.bfloat16)
        for kh in range(3):
            for kw in range(3):
                t = kh * 3 + kw
                off = (kh - 1) * W + (kw - 1)
                if off == 0:
                    shifted = yb
                else:
                    shifted = pltpu.roll(yb, shift=(-off) % HW, axis=0)
                    shifted = shifted * mask_ref[:, t:t + 1]
                t_ref[:, t * Cmid:(t + 1) * Cmid] = shifted
        tv = t_ref[...]
        p0 = jnp.dot(tv, w3b_ref[0], preferred_element_type=jnp.float32)
        p1 = jnp.dot(tv, w3b_ref[1], preferred_element_type=jnp.float32)
        p2 = jnp.dot(tv, w3b_ref[2], preferred_element_type=jnp.float32)
        sv = bn3_ref[0:1] * jax.lax.rsqrt(bn3_ref[3:4] + _EPS)
        bv = bn3_ref[1:2] - bn3_ref[2:3] * sv
        o3_ref[:, 0:Cmid] = jnp.maximum((p1 + p2) * sv + bv, 0.0)
        o3_ref[:, Cmid:2 * Cmid] = jnp.maximum(
            (p0 + p1 + p2) * sv + bv, 0.0)
        o3_ref[:, 2 * Cmid:3 * Cmid] = jnp.maximum((p0 + p1) * sv + bv, 0.0)

    o_ref[0] = jnp.broadcast_to(o3_ref[:, Cmid:2 * Cmid][None],
                                (Dt, HW, Cmid))

    @pl.when(j == 0)
    def _front():
        o_ref[0, 0] = o3_ref[:, 0:Cmid]

    @pl.when(j == D // Dt - 1)
    def _back():
        o_ref[0, Dt - 1] = o3_ref[:, 2 * Cmid:3 * Cmid]


def _linker(x_nchw, w1, bn2d, w3, bn3d):
    N, Cin, H, Wd = x_nchw.shape
    Cmid = w1.shape[1]
    D = H
    HW = H * Wd

    bn2 = jnp.stack(bn2d, axis=0)
    bn3 = jnp.stack(bn3d, axis=0)

    w3r = w3.reshape(3, 9 * Cmid, Cmid)

    x3 = x_nchw.transpose(0, 2, 3, 1).reshape(N, HW, Cin).astype(jnp.float32)
    masks = jnp.asarray(_tap_masks_hw(H, Wd)).astype(jnp.bfloat16)

    Dt = 16
    while D % Dt:
        Dt //= 2

    out = pl.pallas_call(
        functools.partial(_linker_kernel, Wd, D, Cmid, Dt),
        out_shape=jax.ShapeDtypeStruct((N, D, HW, Cmid), jnp.float32),
        grid=(N, D // Dt),
        in_specs=[
            pl.BlockSpec((1, HW, Cin), lambda n, d: (n, 0, 0)),
            pl.BlockSpec((Cin, Cmid), lambda n, d: (0, 0)),
            pl.BlockSpec((4, Cmid), lambda n, d: (0, 0)),
            pl.BlockSpec((HW, 9), lambda n, d: (0, 0)),
            pl.BlockSpec((3, 9 * Cmid, Cmid), lambda n, d: (0, 0, 0)),
            pl.BlockSpec((4, Cmid), lambda n, d: (0, 0)),
        ],
        out_specs=pl.BlockSpec((1, Dt, HW, Cmid), lambda n, d: (n, d, 0, 0)),
        scratch_shapes=[
            pltpu.VMEM((HW, 9 * Cmid), jnp.bfloat16),
            pltpu.VMEM((3, 9 * Cmid, Cmid), jnp.bfloat16),
            pltpu.VMEM((HW, 3 * Cmid), jnp.float32),
        ],
        compiler_params=pltpu.CompilerParams(
            dimension_semantics=("parallel", "arbitrary")),
    )(x3, w1.astype(jnp.float32), bn2, masks, w3r, bn3)

    return out.reshape(N, D, H, Wd, Cmid).transpose(0, 4, 1, 2, 3)


def _base_kernel(x_ref, w_ref, b_ref, o_ref):
    acc = jnp.dot(x_ref[...], w_ref[...], preferred_element_type=jnp.float32)
    o_ref[...] = jnp.maximum(acc + b_ref[...], 0.0)


def _base_matmul(x, w, b, tn=4096):
    M, K = x.shape
    K2, Nc = w.shape
    assert K == K2
    tn = min(tn, Nc)
    if Nc % tn != 0:
        tn = Nc
    b2 = jnp.reshape(b, (1, Nc)).astype(jnp.float32)
    return pl.pallas_call(
        _base_kernel,
        out_shape=jax.ShapeDtypeStruct((M, Nc), jnp.float32),
        grid=(Nc // tn,),
        in_specs=[
            pl.BlockSpec((M, K), lambda j: (0, 0)),
            pl.BlockSpec((K, tn), lambda j: (0, j)),
            pl.BlockSpec((1, tn), lambda j: (0, j)),
        ],
        out_specs=pl.BlockSpec((M, tn), lambda j: (0, j)),
        compiler_params=pltpu.CompilerParams(
            dimension_semantics=("parallel",)),
    )(x.astype(jnp.float32), w.astype(jnp.float32), b2)


def kernel(l0_w1, l0_bn2d_gamma, l0_bn2d_beta, l0_bn2d_mean, l0_bn2d_var,
           l0_w3, l0_bn3d_gamma, l0_bn3d_beta, l0_bn3d_mean, l0_bn3d_var,
           l1_w1, l1_bn2d_gamma, l1_bn2d_beta, l1_bn2d_mean, l1_bn2d_var,
           l1_w3, l1_bn3d_gamma, l1_bn3d_beta, l1_bn3d_mean, l1_bn3d_var,
           base_w, base_b, feat0, feat1, final_vector):
    out0 = _linker(feat0, l0_w1,
                   (l0_bn2d_gamma, l0_bn2d_beta, l0_bn2d_mean, l0_bn2d_var),
                   l0_w3,
                   (l0_bn3d_gamma, l0_bn3d_beta, l0_bn3d_mean, l0_bn3d_var))
    out1 = _linker(feat1, l1_w1,
                   (l1_bn2d_gamma, l1_bn2d_beta, l1_bn2d_mean, l1_bn2d_var),
                   l1_w3,
                   (l1_bn3d_gamma, l1_bn3d_beta, l1_bn3d_mean, l1_bn3d_var))
    N = final_vector.shape[0]
    flat = final_vector.reshape(N, -1)
    x = _base_matmul(flat, base_w, base_b)
    return x, [out0, out1]


# speedup vs baseline: 1.1591x; 1.0638x over previous
import functools

import numpy as np
import jax
import jax.numpy as jnp
from jax.experimental import pallas as pl
from jax.experimental.pallas import tpu as pltpu

_EPS = 1e-5


def _tap_masks_hw(H, W):
    hh = np.arange(H)[:, None]
    ww = np.arange(W)[None, :]
    cols = []
    for kh in range(3):
        for kw in range(3):
            dh, dw = kh - 1, kw - 1
            m = ((hh + dh >= 0) & (hh + dh < H) &
                 (ww + dw >= 0) & (ww + dw < W))
            cols.append(m.reshape(-1))
    return np.stack(cols, axis=1).astype(np.float32)


def _linker_kernel(W, D, Cmid, Dt, x_ref, w1_ref, bn2_ref, mask_ref, w3_ref,
                   bn3_ref, o_ref, t_ref, w3b_ref, o3_ref):
    j = pl.program_id(1)
    HW = x_ref.shape[1]

    @pl.when(j == 0)
    def _convert():
        w3b_ref[...] = w3_ref[...].astype(jnp.bfloat16)

    @pl.when(j == 0)
    def _compute():
        s2 = bn2_ref[0:1] * jax.lax.rsqrt(bn2_ref[3:4] + _EPS)
        b2 = bn2_ref[1:2] - bn2_ref[2:3] * s2
        y = jnp.maximum(
            jnp.dot(x_ref[0], w1_ref[...],
                    preferred_element_type=jnp.float32) * s2 + b2, 0.0)
        yb = y.astype(jnp.bfloat16)
        for kh in range(3):
            for kw in range(3):
                t = kh * 3 + kw
                off = (kh - 1) * W + (kw - 1)
                if off == 0:
                    shifted = yb
                else:
                    shifted = pltpu.roll(yb, shift=(-off) % HW, axis=0)
                    shifted = shifted * mask_ref[:, t:t + 1]
                t_ref[:, t * Cmid:(t + 1) * Cmid] = shifted
        tv = t_ref[...]
        p0 = jnp.dot(tv, w3b_ref[0], preferred_element_type=jnp.float32)
        p1 = jnp.dot(tv, w3b_ref[1], preferred_element_type=jnp.float32)
        p2 = jnp.dot(tv, w3b_ref[2], preferred_element_type=jnp.float32)
        sv = bn3_ref[0:1] * jax.lax.rsqrt(bn3_ref[3:4] + _EPS)
        bv = bn3_ref[1:2] - bn3_ref[2:3] * sv
        o3_ref[:, 0:Cmid] = jnp.maximum((p1 + p2) * sv + bv, 0.0)
        o3_ref[:, Cmid:2 * Cmid] = jnp.maximum(
            (p0 + p1 + p2) * sv + bv, 0.0)
        o3_ref[:, 2 * Cmid:3 * Cmid] = jnp.maximum((p0 + p1) * sv + bv, 0.0)

    o_ref[0] = jnp.broadcast_to(o3_ref[:, Cmid:2 * Cmid][None],
                                (Dt, HW, Cmid))

    @pl.when(j == 0)
    def _front():
        o_ref[0, 0] = o3_ref[:, 0:Cmid]

    @pl.when(j == D // Dt - 1)
    def _back():
        o_ref[0, Dt - 1] = o3_ref[:, 2 * Cmid:3 * Cmid]


def _linker_base_kernel(W, D, Cmid, Dt, x_ref, w1_ref, bn2_ref, mask_ref,
                        w3_ref, bn3_ref, xb_ref, wb_ref, bb_ref,
                        o_ref, ob_ref, t_ref, w3b_ref, o3_ref):
    _linker_kernel(W, D, Cmid, Dt, x_ref, w1_ref, bn2_ref, mask_ref,
                   w3_ref, bn3_ref, o_ref, t_ref, w3b_ref, o3_ref)
    acc = jnp.dot(xb_ref[...], wb_ref[...],
                  preferred_element_type=jnp.float32)
    ob_ref[...] = jnp.maximum(acc + bb_ref[...], 0.0)


def _linker(x_nchw, w1, bn2d, w3, bn3d, base=None):
    N, Cin, H, Wd = x_nchw.shape
    Cmid = w1.shape[1]
    D = H
    HW = H * Wd

    bn2 = jnp.stack(bn2d, axis=0)
    bn3 = jnp.stack(bn3d, axis=0)

    w3r = w3.reshape(3, 9 * Cmid, Cmid)

    x3 = x_nchw.transpose(0, 2, 3, 1).reshape(N, HW, Cin).astype(jnp.float32)
    masks = jnp.asarray(_tap_masks_hw(H, Wd)).astype(jnp.bfloat16)

    Dt = 16
    while D % Dt:
        Dt //= 2

    grid = (N, D // Dt)
    nsteps = N * (D // Dt)
    in_specs = [
        pl.BlockSpec((1, HW, Cin), lambda n, d: (n, 0, 0)),
        pl.BlockSpec((Cin, Cmid), lambda n, d: (0, 0)),
        pl.BlockSpec((4, Cmid), lambda n, d: (0, 0)),
        pl.BlockSpec((HW, 9), lambda n, d: (0, 0)),
        pl.BlockSpec((3, 9 * Cmid, Cmid), lambda n, d: (0, 0, 0)),
        pl.BlockSpec((4, Cmid), lambda n, d: (0, 0)),
    ]
    out_spec = pl.BlockSpec((1, Dt, HW, Cmid), lambda n, d: (n, d, 0, 0))
    scratch = [
        pltpu.VMEM((HW, 9 * Cmid), jnp.bfloat16),
        pltpu.VMEM((3, 9 * Cmid, Cmid), jnp.bfloat16),
        pltpu.VMEM((HW, 3 * Cmid), jnp.float32),
    ]
    args = [x3, w1.astype(jnp.float32), bn2, masks, w3r, bn3]
    cp = pltpu.CompilerParams(
        dimension_semantics=("parallel", "arbitrary"))

    if base is None:
        out = pl.pallas_call(
            functools.partial(_linker_kernel, Wd, D, Cmid, Dt),
            out_shape=jax.ShapeDtypeStruct((N, D, HW, Cmid), jnp.float32),
            grid=grid,
            in_specs=in_specs,
            out_specs=out_spec,
            scratch_shapes=scratch,
            compiler_params=cp,
        )(*args)
        ob = None
    else:
        xb, wb, bb = base
        M, K = xb.shape
        Nc = wb.shape[1]
        tn = Nc // nsteps
        nd = D // Dt
        in_specs = in_specs + [
            pl.BlockSpec((M, K), lambda n, d: (0, 0)),
            pl.BlockSpec((K, tn), lambda n, d: (0, n * nd + d)),
            pl.BlockSpec((1, tn), lambda n, d: (0, n * nd + d)),
        ]
        args = args + [xb.astype(jnp.float32), wb.astype(jnp.float32),
                       jnp.reshape(bb, (1, Nc)).astype(jnp.float32)]
        out, ob = pl.pallas_call(
            functools.partial(_linker_base_kernel, Wd, D, Cmid, Dt),
            out_shape=(
                jax.ShapeDtypeStruct((N, D, HW, Cmid), jnp.float32),
                jax.ShapeDtypeStruct((M, Nc), jnp.float32),
            ),
            grid=grid,
            in_specs=in_specs,
            out_specs=(
                out_spec,
                pl.BlockSpec((M, tn), lambda n, d: (0, n * nd + d)),
            ),
            scratch_shapes=scratch,
            compiler_params=cp,
        )(*args)

    out5 = out.reshape(N, D, H, Wd, Cmid).transpose(0, 4, 1, 2, 3)
    return out5, ob


def _base_kernel(x_ref, w_ref, b_ref, o_ref):
    acc = jnp.dot(x_ref[...], w_ref[...], preferred_element_type=jnp.float32)
    o_ref[...] = jnp.maximum(acc + b_ref[...], 0.0)


def _base_matmul(x, w, b, tn=4096):
    M, K = x.shape
    K2, Nc = w.shape
    assert K == K2
    tn = min(tn, Nc)
    if Nc % tn != 0:
        tn = Nc
    b2 = jnp.reshape(b, (1, Nc)).astype(jnp.float32)
    return pl.pallas_call(
        _base_kernel,
        out_shape=jax.ShapeDtypeStruct((M, Nc), jnp.float32),
        grid=(Nc // tn,),
        in_specs=[
            pl.BlockSpec((M, K), lambda j: (0, 0)),
            pl.BlockSpec((K, tn), lambda j: (0, j)),
            pl.BlockSpec((1, tn), lambda j: (0, j)),
        ],
        out_specs=pl.BlockSpec((M, tn), lambda j: (0, j)),
        compiler_params=pltpu.CompilerParams(
            dimension_semantics=("parallel",)),
    )(x.astype(jnp.float32), w.astype(jnp.float32), b2)


def kernel(l0_w1, l0_bn2d_gamma, l0_bn2d_beta, l0_bn2d_mean, l0_bn2d_var,
           l0_w3, l0_bn3d_gamma, l0_bn3d_beta, l0_bn3d_mean, l0_bn3d_var,
           l1_w1, l1_bn2d_gamma, l1_bn2d_beta, l1_bn2d_mean, l1_bn2d_var,
           l1_w3, l1_bn3d_gamma, l1_bn3d_beta, l1_bn3d_mean, l1_bn3d_var,
           base_w, base_b, feat0, feat1, final_vector):
    N = final_vector.shape[0]
    flat = final_vector.reshape(N, -1)

    D0 = feat0.shape[2]
    Dt0 = 16
    while D0 % Dt0:
        Dt0 //= 2
    nsteps0 = feat0.shape[0] * (D0 // Dt0)
    fuse = base_w.shape[1] % nsteps0 == 0

    out0, x = _linker(feat0, l0_w1,
                      (l0_bn2d_gamma, l0_bn2d_beta, l0_bn2d_mean, l0_bn2d_var),
                      l0_w3,
                      (l0_bn3d_gamma, l0_bn3d_beta, l0_bn3d_mean, l0_bn3d_var),
                      base=(flat, base_w, base_b) if fuse else None)
    out1, _ = _linker(feat1, l1_w1,
                      (l1_bn2d_gamma, l1_bn2d_beta, l1_bn2d_mean, l1_bn2d_var),
                      l1_w3,
                      (l1_bn3d_gamma, l1_bn3d_beta, l1_bn3d_mean, l1_bn3d_var))
    if not fuse:
        x = _base_matmul(flat, base_w, base_b)
    return x, [out0, out1]


# speedup vs baseline: 1.2206x; 1.0531x over previous
import functools

import numpy as np
import jax
import jax.numpy as jnp
from jax.experimental import pallas as pl
from jax.experimental.pallas import tpu as pltpu

_EPS = 1e-5


def _tap_masks_hw(H, W):
    hh = np.arange(H)[:, None]
    ww = np.arange(W)[None, :]
    cols = []
    for kh in range(3):
        for kw in range(3):
            dh, dw = kh - 1, kw - 1
            m = ((hh + dh >= 0) & (hh + dh < H) &
                 (ww + dw >= 0) & (ww + dw < W))
            cols.append(m.reshape(-1))
    return np.stack(cols, axis=1).astype(np.float32)


def _linker_kernel(W, D, Cmid, Dt, x_ref, w1_ref, g2_ref, be2_ref, m2_ref,
                   v2_ref, mask_ref, w3_ref, g3_ref, be3_ref, m3_ref, v3_ref,
                   o_ref, t_ref, w3b_ref, o3_ref):
    j = pl.program_id(1)
    HW = x_ref.shape[1]

    @pl.when(j == 0)
    def _convert():
        w3b_ref[...] = w3_ref[...].astype(jnp.bfloat16)

    @pl.when(j == 0)
    def _compute():
        s2 = g2_ref[...] * jax.lax.rsqrt(v2_ref[...] + _EPS)
        b2 = be2_ref[...] - m2_ref[...] * s2
        y = jnp.maximum(
            jnp.dot(x_ref[0], w1_ref[...],
                    preferred_element_type=jnp.float32) * s2 + b2, 0.0)
        yb = y.astype(jnp.bfloat16)
        for kh in range(3):
            for kw in range(3):
                t = kh * 3 + kw
                off = (kh - 1) * W + (kw - 1)
                if off == 0:
                    shifted = yb
                else:
                    shifted = pltpu.roll(yb, shift=(-off) % HW, axis=0)
                    shifted = shifted * mask_ref[:, t:t + 1]
                t_ref[:, t * Cmid:(t + 1) * Cmid] = shifted
        tv = t_ref[...]
        p0 = jnp.dot(tv, w3b_ref[0], preferred_element_type=jnp.float32)
        p1 = jnp.dot(tv, w3b_ref[1], preferred_element_type=jnp.float32)
        p2 = jnp.dot(tv, w3b_ref[2], preferred_element_type=jnp.float32)
        sv = g3_ref[...] * jax.lax.rsqrt(v3_ref[...] + _EPS)
        bv = be3_ref[...] - m3_ref[...] * sv
        o3_ref[:, 0:Cmid] = jnp.maximum((p1 + p2) * sv + bv, 0.0)
        o3_ref[:, Cmid:2 * Cmid] = jnp.maximum(
            (p0 + p1 + p2) * sv + bv, 0.0)
        o3_ref[:, 2 * Cmid:3 * Cmid] = jnp.maximum((p0 + p1) * sv + bv, 0.0)

    o_ref[0] = jnp.broadcast_to(o3_ref[:, Cmid:2 * Cmid][None],
                                (Dt, HW, Cmid))

    @pl.when(j == 0)
    def _front():
        o_ref[0, 0] = o3_ref[:, 0:Cmid]

    @pl.when(j == D // Dt - 1)
    def _back():
        o_ref[0, Dt - 1] = o3_ref[:, 2 * Cmid:3 * Cmid]


def _linker_base_kernel(W, D, Cmid, Dt, x_ref, w1_ref, g2_ref, be2_ref,
                        m2_ref, v2_ref, mask_ref, w3_ref, g3_ref, be3_ref,
                        m3_ref, v3_ref, xb_ref, wb_ref, bb_ref,
                        o_ref, ob_ref, t_ref, w3b_ref, o3_ref):
    _linker_kernel(W, D, Cmid, Dt, x_ref, w1_ref, g2_ref, be2_ref, m2_ref,
                   v2_ref, mask_ref, w3_ref, g3_ref, be3_ref, m3_ref, v3_ref,
                   o_ref, t_ref, w3b_ref, o3_ref)
    acc = jnp.dot(xb_ref[...], wb_ref[...],
                  preferred_element_type=jnp.float32)
    ob_ref[...] = jnp.maximum(acc + bb_ref[...], 0.0)


def _linker(x_nchw, w1, bn2d, w3, bn3d, base=None):
    N, Cin, H, Wd = x_nchw.shape
    Cmid = w1.shape[1]
    D = H
    HW = H * Wd

    bn2 = [jnp.reshape(v, (1, Cmid)) for v in bn2d]
    bn3 = [jnp.reshape(v, (1, Cmid)) for v in bn3d]

    w3r = w3.reshape(3, 9 * Cmid, Cmid)

    x3 = x_nchw.transpose(0, 2, 3, 1).reshape(N, HW, Cin).astype(jnp.float32)
    masks = jnp.asarray(_tap_masks_hw(H, Wd)).astype(jnp.bfloat16)

    Dt = 16
    while D % Dt:
        Dt //= 2

    grid = (N, D // Dt)
    nsteps = N * (D // Dt)
    vec_spec = pl.BlockSpec((1, Cmid), lambda n, d: (0, 0))
    in_specs = [
        pl.BlockSpec((1, HW, Cin), lambda n, d: (n, 0, 0)),
        pl.BlockSpec((Cin, Cmid), lambda n, d: (0, 0)),
        vec_spec, vec_spec, vec_spec, vec_spec,
        pl.BlockSpec((HW, 9), lambda n, d: (0, 0)),
        pl.BlockSpec((3, 9 * Cmid, Cmid), lambda n, d: (0, 0, 0)),
        vec_spec, vec_spec, vec_spec, vec_spec,
    ]
    out_spec = pl.BlockSpec((1, Dt, HW, Cmid), lambda n, d: (n, d, 0, 0))
    scratch = [
        pltpu.VMEM((HW, 9 * Cmid), jnp.bfloat16),
        pltpu.VMEM((3, 9 * Cmid, Cmid), jnp.bfloat16),
        pltpu.VMEM((HW, 3 * Cmid), jnp.float32),
    ]
    args = [x3, w1.astype(jnp.float32), *bn2, masks, w3r, *bn3]
    cp = pltpu.CompilerParams(
        dimension_semantics=("parallel", "arbitrary"))

    if base is None:
        out = pl.pallas_call(
            functools.partial(_linker_kernel, Wd, D, Cmid, Dt),
            out_shape=jax.ShapeDtypeStruct((N, D, HW, Cmid), jnp.float32),
            grid=grid,
            in_specs=in_specs,
            out_specs=out_spec,
            scratch_shapes=scratch,
            compiler_params=cp,
        )(*args)
        ob = None
    else:
        xb, wb, bb = base
        M, K = xb.shape
        Nc = wb.shape[1]
        tn = Nc // nsteps
        nd = D // Dt
        in_specs = in_specs + [
            pl.BlockSpec((M, K), lambda n, d: (0, 0)),
            pl.BlockSpec((K, tn), lambda n, d: (0, n * nd + d)),
            pl.BlockSpec((1, tn), lambda n, d: (0, n * nd + d)),
        ]
        args = args + [xb.astype(jnp.float32), wb.astype(jnp.float32),
                       jnp.reshape(bb, (1, Nc)).astype(jnp.float32)]
        out, ob = pl.pallas_call(
            functools.partial(_linker_base_kernel, Wd, D, Cmid, Dt),
            out_shape=(
                jax.ShapeDtypeStruct((N, D, HW, Cmid), jnp.float32),
                jax.ShapeDtypeStruct((M, Nc), jnp.float32),
            ),
            grid=grid,
            in_specs=in_specs,
            out_specs=(
                out_spec,
                pl.BlockSpec((M, tn), lambda n, d: (0, n * nd + d)),
            ),
            scratch_shapes=scratch,
            compiler_params=cp,
        )(*args)

    out5 = out.reshape(N, D, H, Wd, Cmid).transpose(0, 4, 1, 2, 3)
    return out5, ob


def _base_kernel(x_ref, w_ref, b_ref, o_ref):
    acc = jnp.dot(x_ref[...], w_ref[...], preferred_element_type=jnp.float32)
    o_ref[...] = jnp.maximum(acc + b_ref[...], 0.0)


def _base_matmul(x, w, b, tn=4096):
    M, K = x.shape
    K2, Nc = w.shape
    assert K == K2
    tn = min(tn, Nc)
    if Nc % tn != 0:
        tn = Nc
    b2 = jnp.reshape(b, (1, Nc)).astype(jnp.float32)
    return pl.pallas_call(
        _base_kernel,
        out_shape=jax.ShapeDtypeStruct((M, Nc), jnp.float32),
        grid=(Nc // tn,),
        in_specs=[
            pl.BlockSpec((M, K), lambda j: (0, 0)),
            pl.BlockSpec((K, tn), lambda j: (0, j)),
            pl.BlockSpec((1, tn), lambda j: (0, j)),
        ],
        out_specs=pl.BlockSpec((M, tn), lambda j: (0, j)),
        compiler_params=pltpu.CompilerParams(
            dimension_semantics=("parallel",)),
    )(x.astype(jnp.float32), w.astype(jnp.float32), b2)


def kernel(l0_w1, l0_bn2d_gamma, l0_bn2d_beta, l0_bn2d_mean, l0_bn2d_var,
           l0_w3, l0_bn3d_gamma, l0_bn3d_beta, l0_bn3d_mean, l0_bn3d_var,
           l1_w1, l1_bn2d_gamma, l1_bn2d_beta, l1_bn2d_mean, l1_bn2d_var,
           l1_w3, l1_bn3d_gamma, l1_bn3d_beta, l1_bn3d_mean, l1_bn3d_var,
           base_w, base_b, feat0, feat1, final_vector):
    N = final_vector.shape[0]
    flat = final_vector.reshape(N, -1)

    D0 = feat0.shape[2]
    Dt0 = 16
    while D0 % Dt0:
        Dt0 //= 2
    nsteps0 = feat0.shape[0] * (D0 // Dt0)
    fuse = base_w.shape[1] % nsteps0 == 0

    out0, x = _linker(feat0, l0_w1,
                      (l0_bn2d_gamma, l0_bn2d_beta, l0_bn2d_mean, l0_bn2d_var),
                      l0_w3,
                      (l0_bn3d_gamma, l0_bn3d_beta, l0_bn3d_mean, l0_bn3d_var),
                      base=(flat, base_w, base_b) if fuse else None)
    out1, _ = _linker(feat1, l1_w1,
                      (l1_bn2d_gamma, l1_bn2d_beta, l1_bn2d_mean, l1_bn2d_var),
                      l1_w3,
                      (l1_bn3d_gamma, l1_bn3d_beta, l1_bn3d_mean, l1_bn3d_var))
    if not fuse:
        x = _base_matmul(flat, base_w, base_b)
    return x, [out0, out1]


# speedup vs baseline: 1.3932x; 1.1414x over previous
import functools

import numpy as np
import jax
import jax.numpy as jnp
from jax.experimental import pallas as pl
from jax.experimental.pallas import tpu as pltpu

_EPS = 1e-5


def _tap_masks_hw(H, W):
    hh = np.arange(H)[:, None]
    ww = np.arange(W)[None, :]
    cols = []
    for kh in range(3):
        for kw in range(3):
            dh, dw = kh - 1, kw - 1
            m = ((hh + dh >= 0) & (hh + dh < H) &
                 (ww + dw >= 0) & (ww + dw < W))
            cols.append(m.reshape(-1))
    return np.stack(cols, axis=1).astype(np.float32)


def _linker_kernel(W, D, Cmid, Dt, x_ref, w1_ref, g2_ref, be2_ref, m2_ref,
                   v2_ref, mask_ref, w3_ref, g3_ref, be3_ref, m3_ref, v3_ref,
                   o_ref, t_ref, w3b_ref, o3_ref):
    j = pl.program_id(1)
    HW = x_ref.shape[1]

    @pl.when(j == 0)
    def _convert():
        w3b_ref[...] = w3_ref[...].astype(jnp.bfloat16)

    @pl.when(j == 0)
    def _compute():
        s2 = g2_ref[...] * jax.lax.rsqrt(v2_ref[...] + _EPS)
        b2 = be2_ref[...] - m2_ref[...] * s2
        y = jnp.maximum(
            jnp.dot(x_ref[0], w1_ref[...],
                    preferred_element_type=jnp.float32) * s2 + b2, 0.0)
        yb = y.astype(jnp.bfloat16)
        for kh in range(3):
            for kw in range(3):
                t = kh * 3 + kw
                off = (kh - 1) * W + (kw - 1)
                if off == 0:
                    shifted = yb
                else:
                    shifted = pltpu.roll(yb, shift=(-off) % HW, axis=0)
                    shifted = shifted * mask_ref[:, t:t + 1]
                t_ref[:, t * Cmid:(t + 1) * Cmid] = shifted
        tv = t_ref[...]
        p0 = jnp.dot(tv, w3b_ref[0], preferred_element_type=jnp.float32)
        p1 = jnp.dot(tv, w3b_ref[1], preferred_element_type=jnp.float32)
        p2 = jnp.dot(tv, w3b_ref[2], preferred_element_type=jnp.float32)
        sv = g3_ref[...] * jax.lax.rsqrt(v3_ref[...] + _EPS)
        bv = be3_ref[...] - m3_ref[...] * sv
        o3_ref[:, 0:Cmid] = jnp.maximum((p1 + p2) * sv + bv, 0.0)
        o3_ref[:, Cmid:2 * Cmid] = jnp.maximum(
            (p0 + p1 + p2) * sv + bv, 0.0)
        o3_ref[:, 2 * Cmid:3 * Cmid] = jnp.maximum((p0 + p1) * sv + bv, 0.0)

    o_ref[0] = jnp.broadcast_to(o3_ref[:, Cmid:2 * Cmid][None],
                                (Dt, HW, Cmid))

    @pl.when(j == 0)
    def _front():
        o_ref[0, 0] = o3_ref[:, 0:Cmid]

    @pl.when(j == D // Dt - 1)
    def _back():
        o_ref[0, Dt - 1] = o3_ref[:, 2 * Cmid:3 * Cmid]


def _linker_base_kernel(W, D, Cmid, Dt, x_ref, w1_ref, g2_ref, be2_ref,
                        m2_ref, v2_ref, mask_ref, w3_ref, g3_ref, be3_ref,
                        m3_ref, v3_ref, xb_ref, wb_ref, bb_ref,
                        o_ref, ob_ref, t_ref, w3b_ref, o3_ref):
    _linker_kernel(W, D, Cmid, Dt, x_ref, w1_ref, g2_ref, be2_ref, m2_ref,
                   v2_ref, mask_ref, w3_ref, g3_ref, be3_ref, m3_ref, v3_ref,
                   o_ref, t_ref, w3b_ref, o3_ref)
    acc = jnp.dot(xb_ref[...], wb_ref[...],
                  preferred_element_type=jnp.float32)
    ob_ref[...] = jnp.maximum(acc + bb_ref[...], 0.0)


def _linker(x_nchw, w1, bn2d, w3, bn3d, base=None):
    N, Cin, H, Wd = x_nchw.shape
    Cmid = w1.shape[1]
    D = H
    HW = H * Wd

    bn2 = [jnp.reshape(v, (1, Cmid)) for v in bn2d]
    bn3 = [jnp.reshape(v, (1, Cmid)) for v in bn3d]

    w3r = w3.reshape(3, 9 * Cmid, Cmid)

    x3 = x_nchw.transpose(0, 2, 3, 1).reshape(N, HW, Cin).astype(jnp.float32)
    masks = jnp.asarray(_tap_masks_hw(H, Wd)).astype(jnp.bfloat16)

    Dt = 32
    while D % Dt:
        Dt //= 2

    grid = (N, D // Dt)
    nsteps = N * (D // Dt)
    vec_spec = pl.BlockSpec((1, Cmid), lambda n, d: (0, 0))
    in_specs = [
        pl.BlockSpec((1, HW, Cin), lambda n, d: (n, 0, 0)),
        pl.BlockSpec((Cin, Cmid), lambda n, d: (0, 0)),
        vec_spec, vec_spec, vec_spec, vec_spec,
        pl.BlockSpec((HW, 9), lambda n, d: (0, 0)),
        pl.BlockSpec((3, 9 * Cmid, Cmid), lambda n, d: (0, 0, 0)),
        vec_spec, vec_spec, vec_spec, vec_spec,
    ]
    out_spec = pl.BlockSpec((1, Dt, HW, Cmid), lambda n, d: (n, d, 0, 0))
    scratch = [
        pltpu.VMEM((HW, 9 * Cmid), jnp.bfloat16),
        pltpu.VMEM((3, 9 * Cmid, Cmid), jnp.bfloat16),
        pltpu.VMEM((HW, 3 * Cmid), jnp.float32),
    ]
    args = [x3, w1.astype(jnp.float32), *bn2, masks, w3r, *bn3]
    cp = pltpu.CompilerParams(
        dimension_semantics=("parallel", "arbitrary"))

    if base is None:
        out = pl.pallas_call(
            functools.partial(_linker_kernel, Wd, D, Cmid, Dt),
            out_shape=jax.ShapeDtypeStruct((N, D, HW, Cmid), jnp.float32),
            grid=grid,
            in_specs=in_specs,
            out_specs=out_spec,
            scratch_shapes=scratch,
            compiler_params=cp,
        )(*args)
        ob = None
    else:
        xb, wb, bb = base
        M, K = xb.shape
        Nc = wb.shape[1]
        tn = Nc // nsteps
        nd = D // Dt
        in_specs = in_specs + [
            pl.BlockSpec((M, K), lambda n, d: (0, 0)),
            pl.BlockSpec((K, tn), lambda n, d: (0, n * nd + d)),
            pl.BlockSpec((1, tn), lambda n, d: (0, n * nd + d)),
        ]
        args = args + [xb.astype(jnp.float32), wb.astype(jnp.float32),
                       jnp.reshape(bb, (1, Nc)).astype(jnp.float32)]
        out, ob = pl.pallas_call(
            functools.partial(_linker_base_kernel, Wd, D, Cmid, Dt),
            out_shape=(
                jax.ShapeDtypeStruct((N, D, HW, Cmid), jnp.float32),
                jax.ShapeDtypeStruct((M, Nc), jnp.float32),
            ),
            grid=grid,
            in_specs=in_specs,
            out_specs=(
                out_spec,
                pl.BlockSpec((M, tn), lambda n, d: (0, n * nd + d)),
            ),
            scratch_shapes=scratch,
            compiler_params=cp,
        )(*args)

    out5 = out.reshape(N, D, H, Wd, Cmid).transpose(0, 4, 1, 2, 3)
    return out5, ob


def _base_kernel(x_ref, w_ref, b_ref, o_ref):
    acc = jnp.dot(x_ref[...], w_ref[...], preferred_element_type=jnp.float32)
    o_ref[...] = jnp.maximum(acc + b_ref[...], 0.0)


def _base_matmul(x, w, b, tn=4096):
    M, K = x.shape
    K2, Nc = w.shape
    assert K == K2
    tn = min(tn, Nc)
    if Nc % tn != 0:
        tn = Nc
    b2 = jnp.reshape(b, (1, Nc)).astype(jnp.float32)
    return pl.pallas_call(
        _base_kernel,
        out_shape=jax.ShapeDtypeStruct((M, Nc), jnp.float32),
        grid=(Nc // tn,),
        in_specs=[
            pl.BlockSpec((M, K), lambda j: (0, 0)),
            pl.BlockSpec((K, tn), lambda j: (0, j)),
            pl.BlockSpec((1, tn), lambda j: (0, j)),
        ],
        out_specs=pl.BlockSpec((M, tn), lambda j: (0, j)),
        compiler_params=pltpu.CompilerParams(
            dimension_semantics=("parallel",)),
    )(x.astype(jnp.float32), w.astype(jnp.float32), b2)


def kernel(l0_w1, l0_bn2d_gamma, l0_bn2d_beta, l0_bn2d_mean, l0_bn2d_var,
           l0_w3, l0_bn3d_gamma, l0_bn3d_beta, l0_bn3d_mean, l0_bn3d_var,
           l1_w1, l1_bn2d_gamma, l1_bn2d_beta, l1_bn2d_mean, l1_bn2d_var,
           l1_w3, l1_bn3d_gamma, l1_bn3d_beta, l1_bn3d_mean, l1_bn3d_var,
           base_w, base_b, feat0, feat1, final_vector):
    N = final_vector.shape[0]
    flat = final_vector.reshape(N, -1)

    D0 = feat0.shape[2]
    Dt0 = 32
    while D0 % Dt0:
        Dt0 //= 2
    nsteps0 = feat0.shape[0] * (D0 // Dt0)
    fuse = base_w.shape[1] % nsteps0 == 0

    out0, x = _linker(feat0, l0_w1,
                      (l0_bn2d_gamma, l0_bn2d_beta, l0_bn2d_mean, l0_bn2d_var),
                      l0_w3,
                      (l0_bn3d_gamma, l0_bn3d_beta, l0_bn3d_mean, l0_bn3d_var),
                      base=(flat, base_w, base_b) if fuse else None)
    out1, _ = _linker(feat1, l1_w1,
                      (l1_bn2d_gamma, l1_bn2d_beta, l1_bn2d_mean, l1_bn2d_var),
                      l1_w3,
                      (l1_bn3d_gamma, l1_bn3d_beta, l1_bn3d_mean, l1_bn3d_var))
    if not fuse:
        x = _base_matmul(flat, base_w, base_b)
    return x, [out0, out1]
